# baseline (speedup 1.0000x reference)
import jax
import jax.numpy as jnp
from jax import lax
from jax.experimental import pallas as pl
from jax.experimental.pallas import tpu as pltpu
from jax.experimental.pallas import tpu_sc as plsc

N = 10000
E = 160000
P = E // 2
P2 = P // 2
NCORES = 2
NSUB = 16
NW = NCORES * NSUB
PPW = P2 // NW
CH = 125
NCH = PPW // CH
SUP = 5
NSUP = NCH // SUP
ROWS = CH * SUP
NPT = 624

_SC_MESH = plsc.VectorSubcoreMesh(
    core_axis_name="c", subcore_axis_name="s",
    num_cores=NCORES, num_subcores=NSUB)
_SC_PARAMS = pltpu.CompilerParams(use_tc_tiling_on_sc=False)


def _gather_h_body(h_hbm, se_hbm, de_hbm, so_hbm, do_hbm, g_hbm,
                   idx_se, idx_de, idx_so, idx_do,
                   buf_se, buf_de, buf_so, buf_do, table, sem):
    cid = lax.axis_index("c")
    sid = lax.axis_index("s")
    wid = sid * NCORES + cid
    pltpu.sync_copy(h_hbm.at[pl.ds(sid * NPT, NPT)],
                    table.at[pl.ds(sid * NPT, NPT)])

    @pl.when(sid == NSUB - 1)
    def _tail_stage():
        pltpu.sync_copy(h_hbm.at[pl.ds(NSUB * NPT, N - NSUB * NPT)],
                        table.at[pl.ds(NSUB * NPT, N - NSUB * NPT)])

    pltpu.sync_copy(se_hbm.at[wid], idx_se)
    pltpu.sync_copy(de_hbm.at[wid], idx_de)
    pltpu.sync_copy(so_hbm.at[wid], idx_so)
    pltpu.sync_copy(do_hbm.at[wid], idx_do)
    plsc.subcore_barrier()

    @pl.loop(0, NSUP)
    def _sup(sc):
        base = wid * PPW + sc * ROWS
        descs = []
        for k in range(SUP):
            j = sc * SUP + k
            sl = pl.ds(k * CH, CH)
            descs.append(pltpu.async_copy(
                table.at[idx_se.at[j]], buf_se.at[sl], sem))
            descs.append(pltpu.async_copy(
                table.at[idx_de.at[j]], buf_de.at[sl], sem))
            descs.append(pltpu.async_copy(
                table.at[idx_so.at[j]], buf_so.at[sl], sem))
            descs.append(pltpu.async_copy(
                table.at[idx_do.at[j]], buf_do.at[sl], sem))
        for d in descs:
            d.wait()
        pltpu.sync_copy(buf_se, g_hbm.at[pl.ds(base, ROWS), pl.ds(0, 32)])
        pltpu.sync_copy(buf_de, g_hbm.at[pl.ds(base, ROWS), pl.ds(32, 32)])
        pltpu.sync_copy(buf_so, g_hbm.at[pl.ds(base, ROWS), pl.ds(64, 32)])
        pltpu.sync_copy(buf_do, g_hbm.at[pl.ds(base, ROWS), pl.ds(96, 32)])


def _gather_ts_body(ts_hbm, se_hbm, de_hbm, so_hbm, do_hbm, tg_hbm,
                    idx_se, idx_de, idx_so, idx_do,
                    buf_se, buf_de, buf_so, buf_do, table, sem):
    cid = lax.axis_index("c")
    sid = lax.axis_index("s")
    wid = sid * NCORES + cid
    pltpu.sync_copy(ts_hbm.at[pl.ds(sid * NPT, NPT)],
                    table.at[pl.ds(sid * NPT, NPT)])

    @pl.when(sid == NSUB - 1)
    def _tail_stage():
        pltpu.sync_copy(ts_hbm.at[pl.ds(NSUB * NPT, N - NSUB * NPT)],
                        table.at[pl.ds(NSUB * NPT, N - NSUB * NPT)])

    pltpu.sync_copy(se_hbm.at[wid], idx_se)
    pltpu.sync_copy(de_hbm.at[wid], idx_de)
    pltpu.sync_copy(so_hbm.at[wid], idx_so)
    pltpu.sync_copy(do_hbm.at[wid], idx_do)
    plsc.subcore_barrier()

    @pl.loop(0, NSUP)
    def _sup(sc):
        base = wid * PPW + sc * ROWS
        descs = []
        for k in range(SUP):
            j = sc * SUP + k
            sl = pl.ds(k * CH, CH)
            descs.append(pltpu.async_copy(
                table.at[idx_se.at[j]], buf_se.at[sl], sem))
            descs.append(pltpu.async_copy(
                table.at[idx_de.at[j]], buf_de.at[sl], sem))
            descs.append(pltpu.async_copy(
                table.at[idx_so.at[j]], buf_so.at[sl], sem))
            descs.append(pltpu.async_copy(
                table.at[idx_do.at[j]], buf_do.at[sl], sem))
        for d in descs:
            d.wait()
        pltpu.sync_copy(buf_se, tg_hbm.at[pl.ds(base, ROWS), pl.ds(0, 16)])
        pltpu.sync_copy(buf_de, tg_hbm.at[pl.ds(base, ROWS), pl.ds(16, 16)])
        pltpu.sync_copy(buf_so, tg_hbm.at[pl.ds(base, ROWS), pl.ds(32, 16)])
        pltpu.sync_copy(buf_do, tg_hbm.at[pl.ds(base, ROWS), pl.ds(48, 16)])


def _scatter_body(ea_hbm, de_hbm, do_hbm, zeros_hbm, agg_hbm,
                  idx_de, idx_do, ebuf_e, ebuf_o, acc, sem):
    cid = lax.axis_index("c")
    sid = lax.axis_index("s")
    wid = sid * NCORES + cid
    pltpu.sync_copy(zeros_hbm.at[pl.ds(sid * NPT, NPT)],
                    acc.at[pl.ds(sid * NPT, NPT)])

    @pl.when(sid == NSUB - 1)
    def _tail_zero():
        pltpu.sync_copy(zeros_hbm.at[pl.ds(NSUB * NPT, N - NSUB * NPT)],
                        acc.at[pl.ds(NSUB * NPT, N - NSUB * NPT)])

    pltpu.sync_copy(de_hbm.at[wid], idx_de)
    pltpu.sync_copy(do_hbm.at[wid], idx_do)
    plsc.subcore_barrier()

    @pl.loop(0, NSUP)
    def _sup(sc):
        base = wid * PPW + sc * ROWS
        pltpu.sync_copy(ea_hbm.at[pl.ds(base, ROWS), pl.ds(0, 64)], ebuf_e)
        pltpu.sync_copy(ea_hbm.at[pl.ds(base, ROWS), pl.ds(64, 64)], ebuf_o)
        for k in range(SUP):
            j = sc * SUP + k
            sl = pl.ds(k * CH, CH)
            pltpu.sync_copy(ebuf_e.at[sl], acc.at[idx_de.at[j]], add=True)
            pltpu.sync_copy(ebuf_o.at[sl], acc.at[idx_do.at[j]], add=True)

    plsc.subcore_barrier()
    pltpu.sync_copy(acc.at[pl.ds(sid * NPT, NPT)],
                    agg_hbm.at[pl.ds(sid * NPT, NPT), pl.ds(cid * 64, 64)])

    @pl.when(sid == NSUB - 1)
    def _tail_out():
        pltpu.sync_copy(acc.at[pl.ds(NSUB * NPT, N - NSUB * NPT)],
                        agg_hbm.at[pl.ds(NSUB * NPT, N - NSUB * NPT),
                                   pl.ds(cid * 64, 64)])


def _sc_gather_h(h, se, de, so, do):
    return pl.kernel(
        _gather_h_body,
        out_type=jax.ShapeDtypeStruct((P2, 128), jnp.float32),
        mesh=_SC_MESH,
        compiler_params=_SC_PARAMS,
        scratch_types=[
            pltpu.VMEM((NCH, CH), jnp.int32),
            pltpu.VMEM((NCH, CH), jnp.int32),
            pltpu.VMEM((NCH, CH), jnp.int32),
            pltpu.VMEM((NCH, CH), jnp.int32),
            pltpu.VMEM((ROWS, 32), jnp.float32),
            pltpu.VMEM((ROWS, 32), jnp.float32),
            pltpu.VMEM((ROWS, 32), jnp.float32),
            pltpu.VMEM((ROWS, 32), jnp.float32),
            pltpu.VMEM_SHARED((N, 32), jnp.float32),
            pltpu.SemaphoreType.DMA,
        ],
    )(h, se, de, so, do)


def _sc_gather_ts(ts16, se, de, so, do):
    return pl.kernel(
        _gather_ts_body,
        out_type=jax.ShapeDtypeStruct((P2, 64), jnp.float32),
        mesh=_SC_MESH,
        compiler_params=_SC_PARAMS,
        scratch_types=[
            pltpu.VMEM((NCH, CH), jnp.int32),
            pltpu.VMEM((NCH, CH), jnp.int32),
            pltpu.VMEM((NCH, CH), jnp.int32),
            pltpu.VMEM((NCH, CH), jnp.int32),
            pltpu.VMEM((ROWS, 16), jnp.float32),
            pltpu.VMEM((ROWS, 16), jnp.float32),
            pltpu.VMEM((ROWS, 16), jnp.float32),
            pltpu.VMEM((ROWS, 16), jnp.float32),
            pltpu.VMEM_SHARED((N, 16), jnp.float32),
            pltpu.SemaphoreType.DMA,
        ],
    )(ts16, se, de, so, do)


def _sc_scatter(ea2, de, do, zeros_n64):
    return pl.kernel(
        _scatter_body,
        out_type=jax.ShapeDtypeStruct((N, 128), jnp.float32),
        mesh=_SC_MESH,
        compiler_params=_SC_PARAMS,
        scratch_types=[
            pltpu.VMEM((NCH, CH), jnp.int32),
            pltpu.VMEM((NCH, CH), jnp.int32),
            pltpu.VMEM((ROWS, 64), jnp.float32),
            pltpu.VMEM((ROWS, 64), jnp.float32),
            pltpu.VMEM_SHARED((N, 64), jnp.float32),
            pltpu.SemaphoreType.DMA,
        ],
    )(ea2, de, do, zeros_n64)


BN = 1000
BP = 4000


def _enc_node_body(x_ref, w1, b1, w2, b2, wi, bi, h_ref, cn_ref):
    h1 = jnp.maximum(x_ref[...] @ w1[...] + b1[...], 0.0)
    h = h1 @ w2[...] + b2[...]
    h_ref[...] = h
    cn_ref[...] = h @ wi[...] + bi[...]


def _enc_edge_body(a2_ref, tg_ref, w1, b1, w2, b2, ea_ref, dt_ref):
    a2 = a2_ref[...]
    h1e = jnp.maximum(a2[:, 0:8] @ w1[...] + b1[...], 0.0)
    h1o = jnp.maximum(a2[:, 8:16] @ w1[...] + b1[...], 0.0)
    ea_ref[...] = jnp.concatenate(
        [h1e @ w2[...] + b2[...], h1o @ w2[...] + b2[...]], axis=1)
    tg = tg_ref[...]
    dt_ref[...] = jnp.concatenate(
        [tg[:, 16:17] - tg[:, 0:1], tg[:, 48:49] - tg[:, 32:33]], axis=1)


def _edge_mlp_body(g_ref, ea_ref, dt_ref, w1sde, wdt, b1, w2, b2,
                   out_ref):
    g = g_ref[...]
    ea = ea_ref[...]
    dt = dt_ref[...]
    w1 = w1sde[...].astype(jnp.bfloat16)
    w2b = w2[...].astype(jnp.bfloat16)
    f_e = jnp.concatenate([g[:, 0:64], ea[:, 0:64]], axis=1).astype(
        jnp.bfloat16)
    f_o = jnp.concatenate([g[:, 64:128], ea[:, 64:128]], axis=1).astype(
        jnp.bfloat16)
    mm = lambda a, b: jax.lax.dot(a, b, preferred_element_type=jnp.float32)
    t_e = mm(f_e, w1) + dt[:, 0:1] * wdt[...] + b1[...]
    t_o = mm(f_o, w1) + dt[:, 1:2] * wdt[...] + b1[...]
    u_e = jnp.maximum(t_e, 0.0).astype(jnp.bfloat16)
    u_o = jnp.maximum(t_o, 0.0).astype(jnp.bfloat16)
    out_ref[...] = jnp.concatenate(
        [mm(u_e, w2b) + b2[...], mm(u_o, w2b) + b2[...]], axis=1)


def _node_mlp_body(h_ref, agga_ref, aggb_ref, cn_ref, wh, wa, w2, b2,
                   out_ref):
    a = agga_ref[...]
    b = aggb_ref[...]
    agg = (a[:, 0:64] + a[:, 64:128]) + (b[:, 0:64] + b[:, 64:128])
    pre = h_ref[...] @ wh[...] + agg @ wa[...] + cn_ref[...]
    out_ref[...] = jnp.maximum(pre, 0.0) @ w2[...] + b2[...]


def _cls_body(ea_ref, w1, b1, w2, b2, out_ref):
    ea = ea_ref[...]
    u_e = jnp.maximum(ea[:, 0:64] @ w1[...] + b1[...], 0.0)
    u_o = jnp.maximum(ea[:, 64:128] @ w1[...] + b1[...], 0.0)
    out_ref[...] = jax.nn.sigmoid(jnp.concatenate(
        [u_e @ w2[...] + b2[...], u_o @ w2[...] + b2[...]], axis=1))


def _full(shape):
    return pl.BlockSpec(shape, lambda i: tuple(0 for _ in shape))


def _tc_enc_node(x, ne_W1, ne_b1, ne_W2, ne_b2, nmWi, nm_b1):
    return pl.pallas_call(
        _enc_node_body,
        grid=(N // BN,),
        in_specs=[pl.BlockSpec((BN, 512), lambda i: (i, 0)),
                  _full((512, 256)), _full((1, 256)),
                  _full((256, 32)), _full((1, 32)),
                  _full((32, 32)), _full((1, 32))],
        out_specs=[pl.BlockSpec((BN, 32), lambda i: (i, 0)),
                   pl.BlockSpec((BN, 32), lambda i: (i, 0))],
        out_shape=[jax.ShapeDtypeStruct((N, 32), jnp.float32),
                   jax.ShapeDtypeStruct((N, 32), jnp.float32)],
    )(x, ne_W1, ne_b1, ne_W2, ne_b2, nmWi, nm_b1)


def _tc_enc_edge(attr2, tg, ee_W1p, ee_b1, ee_W2, ee_b2):
    return pl.pallas_call(
        _enc_edge_body,
        grid=(P2 // BP,),
        in_specs=[pl.BlockSpec((BP, 16), lambda i: (i, 0)),
                  pl.BlockSpec((BP, 64), lambda i: (i, 0)),
                  _full((8, 32)), _full((1, 32)),
                  _full((32, 64)), _full((1, 64))],
        out_specs=[pl.BlockSpec((BP, 128), lambda i: (i, 0)),
                   pl.BlockSpec((BP, 2), lambda i: (i, 0))],
        out_shape=[jax.ShapeDtypeStruct((P2, 128), jnp.float32),
                   jax.ShapeDtypeStruct((P2, 2), jnp.float32)],
    )(attr2, tg, ee_W1p, ee_b1, ee_W2, ee_b2)


def _tc_edge_mlp(g2, ea2, dt2, w1sde, wdt, b1, w2, b2):
    return pl.pallas_call(
        _edge_mlp_body,
        grid=(P2 // BP,),
        in_specs=[pl.BlockSpec((BP, 128), lambda i: (i, 0)),
                  pl.BlockSpec((BP, 128), lambda i: (i, 0)),
                  pl.BlockSpec((BP, 2), lambda i: (i, 0)),
                  _full((128, 64)), _full((1, 64)),
                  _full((1, 64)), _full((64, 64)), _full((1, 64))],
        out_specs=pl.BlockSpec((BP, 128), lambda i: (i, 0)),
        out_shape=jax.ShapeDtypeStruct((P2, 128), jnp.float32),
    )(g2, ea2, dt2, w1sde, wdt, b1, w2, b2)


def _tc_node_mlp(h, agga, aggb, cn, wh, wa, w2, b2):
    return pl.pallas_call(
        _node_mlp_body,
        grid=(N // BN,),
        in_specs=[pl.BlockSpec((BN, 32), lambda i: (i, 0)),
                  pl.BlockSpec((BN, 128), lambda i: (i, 0)),
                  pl.BlockSpec((BN, 128), lambda i: (i, 0)),
                  pl.BlockSpec((BN, 32), lambda i: (i, 0)),
                  _full((32, 32)), _full((64, 32)),
                  _full((32, 32)), _full((1, 32))],
        out_specs=pl.BlockSpec((BN, 32), lambda i: (i, 0)),
        out_shape=jax.ShapeDtypeStruct((N, 32), jnp.float32),
    )(h, agga, aggb, cn, wh, wa, w2, b2)


def _tc_cls(ea2, ec_W1, ec_b1, ec_W2, ec_b2):
    return pl.pallas_call(
        _cls_body,
        grid=(P2 // BP,),
        in_specs=[pl.BlockSpec((BP, 128), lambda i: (i, 0)),
                  _full((64, 64)), _full((1, 64)),
                  _full((64, 1)), _full((1, 1))],
        out_specs=pl.BlockSpec((BP, 2), lambda i: (i, 0)),
        out_shape=jax.ShapeDtypeStruct((P2, 2), jnp.float32),
    )(ea2, ec_W1, ec_b1, ec_W2, ec_b2)


def kernel(x, edge_index, edge_attr, node_timestamps, batch, initial_x,
           ee_W1, ee_b1, ee_W2, ee_b2,
           ne_W1, ne_b1, ne_W2, ne_b2,
           em_W1, em_b1, em_W2, em_b2,
           nm_W1, nm_b1, nm_W2, nm_b2,
           ec_W1, ec_b1, ec_W2, ec_b2):
    src = edge_index[0]
    dst = edge_index[1]
    rsh = lambda a: a.reshape(NW, NCH, CH)
    idx_a = (rsh(src[:P2]), rsh(dst[:P2]),
             rsh(src[P:P + P2]), rsh(dst[P:P + P2]))
    idx_b = (rsh(src[P2:P]), rsh(dst[P2:P]),
             rsh(src[P + P2:]), rsh(dst[P + P2:]))

    attr_p = jnp.pad(edge_attr, ((0, 0), (0, 2)))
    attr2 = jnp.concatenate([attr_p[:P], attr_p[P:]], axis=1)
    attr2a, attr2b = attr2[:P2], attr2[P2:]
    ee_W1p = jnp.pad(ee_W1, ((0, 2), (0, 0)))
    ts16 = jnp.pad(node_timestamps[:, None], ((0, 0), (0, 15)))
    w1sde = em_W1[0:128]
    wdt = em_W1[128:129]
    nmWh = nm_W1[0:32]
    nmWa = nm_W1[32:96]
    nmWi = nm_W1[96:128]
    zeros_n64 = jnp.zeros((N, 64), jnp.float32)

    r = lambda b: b[None, :]

    h, cn = _tc_enc_node(x, ne_W1, r(ne_b1), ne_W2, r(ne_b2), nmWi, r(nm_b1))
    tga = _sc_gather_ts(ts16, *idx_a)
    tgb = _sc_gather_ts(ts16, *idx_b)
    ea2a, dt2a = _tc_enc_edge(attr2a, tga, ee_W1p, r(ee_b1), ee_W2, r(ee_b2))
    ea2b, dt2b = _tc_enc_edge(attr2b, tgb, ee_W1p, r(ee_b1), ee_W2, r(ee_b2))

    for _ in range(12):
        g2a = _sc_gather_h(h, *idx_a)
        g2b = _sc_gather_h(h, *idx_b)
        ea2a = _tc_edge_mlp(g2a, ea2a, dt2a, w1sde, wdt, r(em_b1), em_W2,
                            r(em_b2))
        agga = _sc_scatter(ea2a, idx_a[1], idx_a[3], zeros_n64)
        ea2b = _tc_edge_mlp(g2b, ea2b, dt2b, w1sde, wdt, r(em_b1), em_W2,
                            r(em_b2))
        aggb = _sc_scatter(ea2b, idx_b[1], idx_b[3], zeros_n64)
        h = _tc_node_mlp(h, agga, aggb, cn, nmWh, nmWa, nm_W2, r(nm_b2))

    outa = _tc_cls(ea2a, ec_W1, r(ec_b1), ec_W2, r(ec_b2))
    outb = _tc_cls(ea2b, ec_W1, r(ec_b1), ec_W2, r(ec_b2))
    return jnp.concatenate([outa[:, 0:1], outb[:, 0:1],
                            outa[:, 1:2], outb[:, 1:2]], axis=0)


# speedup vs baseline: 1.0287x; 1.0287x over previous
import jax
import jax.numpy as jnp
from jax import lax
from jax.experimental import pallas as pl
from jax.experimental.pallas import tpu as pltpu
from jax.experimental.pallas import tpu_sc as plsc

N = 10000
E = 160000
P = E // 2
P2 = P // 2
NCORES = 2
NSUB = 16
NW = NCORES * NSUB
PPW = P2 // NW
CH = 125
NCH = PPW // CH
SUP = 5
NSUP = NCH // SUP
ROWS = CH * SUP
NPT = 624

_SC_MESH = plsc.VectorSubcoreMesh(
    core_axis_name="c", subcore_axis_name="s",
    num_cores=NCORES, num_subcores=NSUB)
_SC_PARAMS = pltpu.CompilerParams(use_tc_tiling_on_sc=False)


def _gather_h_body(h_hbm, se_hbm, de_hbm, so_hbm, do_hbm, g_hbm,
                   idx_se, idx_de, idx_so, idx_do,
                   buf_se, buf_de, buf_so, buf_do, table, sem):
    cid = lax.axis_index("c")
    sid = lax.axis_index("s")
    wid = sid * NCORES + cid
    pre = [pltpu.async_copy(h_hbm.at[pl.ds(sid * NPT, NPT)],
                            table.at[pl.ds(sid * NPT, NPT)], sem),
           pltpu.async_copy(se_hbm.at[wid], idx_se, sem),
           pltpu.async_copy(de_hbm.at[wid], idx_de, sem),
           pltpu.async_copy(so_hbm.at[wid], idx_so, sem),
           pltpu.async_copy(do_hbm.at[wid], idx_do, sem)]
    for d in pre:
        d.wait()

    @pl.when(sid == NSUB - 1)
    def _tail_stage():
        pltpu.sync_copy(h_hbm.at[pl.ds(NSUB * NPT, N - NSUB * NPT)],
                        table.at[pl.ds(NSUB * NPT, N - NSUB * NPT)])

    plsc.subcore_barrier()

    @pl.loop(0, NSUP)
    def _sup(sc):
        base = wid * PPW + sc * ROWS
        descs = []
        for k in range(SUP):
            j = sc * SUP + k
            sl = pl.ds(k * CH, CH)
            descs.append(pltpu.async_copy(
                table.at[idx_se.at[j]], buf_se.at[sl], sem))
            descs.append(pltpu.async_copy(
                table.at[idx_de.at[j]], buf_de.at[sl], sem))
            descs.append(pltpu.async_copy(
                table.at[idx_so.at[j]], buf_so.at[sl], sem))
            descs.append(pltpu.async_copy(
                table.at[idx_do.at[j]], buf_do.at[sl], sem))
        for d in descs:
            d.wait()
        pltpu.sync_copy(buf_se, g_hbm.at[pl.ds(base, ROWS), pl.ds(0, 32)])
        pltpu.sync_copy(buf_de, g_hbm.at[pl.ds(base, ROWS), pl.ds(32, 32)])
        pltpu.sync_copy(buf_so, g_hbm.at[pl.ds(base, ROWS), pl.ds(64, 32)])
        pltpu.sync_copy(buf_do, g_hbm.at[pl.ds(base, ROWS), pl.ds(96, 32)])


def _gather_ts_body(ts_hbm, se_hbm, de_hbm, so_hbm, do_hbm, tg_hbm,
                    idx_se, idx_de, idx_so, idx_do,
                    buf_se, buf_de, buf_so, buf_do, table, sem):
    cid = lax.axis_index("c")
    sid = lax.axis_index("s")
    wid = sid * NCORES + cid
    pre = [pltpu.async_copy(ts_hbm.at[pl.ds(sid * NPT, NPT)],
                            table.at[pl.ds(sid * NPT, NPT)], sem),
           pltpu.async_copy(se_hbm.at[wid], idx_se, sem),
           pltpu.async_copy(de_hbm.at[wid], idx_de, sem),
           pltpu.async_copy(so_hbm.at[wid], idx_so, sem),
           pltpu.async_copy(do_hbm.at[wid], idx_do, sem)]
    for d in pre:
        d.wait()

    @pl.when(sid == NSUB - 1)
    def _tail_stage():
        pltpu.sync_copy(ts_hbm.at[pl.ds(NSUB * NPT, N - NSUB * NPT)],
                        table.at[pl.ds(NSUB * NPT, N - NSUB * NPT)])

    plsc.subcore_barrier()

    @pl.loop(0, NSUP)
    def _sup(sc):
        base = wid * PPW + sc * ROWS
        descs = []
        for k in range(SUP):
            j = sc * SUP + k
            sl = pl.ds(k * CH, CH)
            descs.append(pltpu.async_copy(
                table.at[idx_se.at[j]], buf_se.at[sl], sem))
            descs.append(pltpu.async_copy(
                table.at[idx_de.at[j]], buf_de.at[sl], sem))
            descs.append(pltpu.async_copy(
                table.at[idx_so.at[j]], buf_so.at[sl], sem))
            descs.append(pltpu.async_copy(
                table.at[idx_do.at[j]], buf_do.at[sl], sem))
        for d in descs:
            d.wait()
        pltpu.sync_copy(buf_se, tg_hbm.at[pl.ds(base, ROWS), pl.ds(0, 16)])
        pltpu.sync_copy(buf_de, tg_hbm.at[pl.ds(base, ROWS), pl.ds(16, 16)])
        pltpu.sync_copy(buf_so, tg_hbm.at[pl.ds(base, ROWS), pl.ds(32, 16)])
        pltpu.sync_copy(buf_do, tg_hbm.at[pl.ds(base, ROWS), pl.ds(48, 16)])


def _scatter_body(ea_hbm, de_hbm, do_hbm, zeros_hbm, agg_hbm,
                  idx_de, idx_do, ebuf_e, ebuf_o, acc, sem):
    cid = lax.axis_index("c")
    sid = lax.axis_index("s")
    wid = sid * NCORES + cid
    pre = [pltpu.async_copy(zeros_hbm.at[pl.ds(sid * NPT, NPT)],
                            acc.at[pl.ds(sid * NPT, NPT)], sem),
           pltpu.async_copy(de_hbm.at[wid], idx_de, sem),
           pltpu.async_copy(do_hbm.at[wid], idx_do, sem)]
    for d in pre:
        d.wait()

    @pl.when(sid == NSUB - 1)
    def _tail_zero():
        pltpu.sync_copy(zeros_hbm.at[pl.ds(NSUB * NPT, N - NSUB * NPT)],
                        acc.at[pl.ds(NSUB * NPT, N - NSUB * NPT)])

    plsc.subcore_barrier()

    @pl.loop(0, NSUP)
    def _sup(sc):
        base = wid * PPW + sc * ROWS
        lds = [pltpu.async_copy(
                   ea_hbm.at[pl.ds(base, ROWS), pl.ds(0, 64)], ebuf_e, sem),
               pltpu.async_copy(
                   ea_hbm.at[pl.ds(base, ROWS), pl.ds(64, 64)], ebuf_o, sem)]
        for d in lds:
            d.wait()
        descs = []
        for k in range(SUP):
            j = sc * SUP + k
            sl = pl.ds(k * CH, CH)
            descs.append(pltpu.async_copy(
                ebuf_e.at[sl], acc.at[idx_de.at[j]], sem, add=True))
            descs.append(pltpu.async_copy(
                ebuf_o.at[sl], acc.at[idx_do.at[j]], sem, add=True))
        for d in descs:
            d.wait()

    plsc.subcore_barrier()
    pltpu.sync_copy(acc.at[pl.ds(sid * NPT, NPT)],
                    agg_hbm.at[pl.ds(sid * NPT, NPT), pl.ds(cid * 64, 64)])

    @pl.when(sid == NSUB - 1)
    def _tail_out():
        pltpu.sync_copy(acc.at[pl.ds(NSUB * NPT, N - NSUB * NPT)],
                        agg_hbm.at[pl.ds(NSUB * NPT, N - NSUB * NPT),
                                   pl.ds(cid * 64, 64)])


def _sc_gather_h(h, se, de, so, do):
    return pl.kernel(
        _gather_h_body,
        out_type=jax.ShapeDtypeStruct((P2, 128), jnp.float32),
        mesh=_SC_MESH,
        compiler_params=_SC_PARAMS,
        scratch_types=[
            pltpu.VMEM((NCH, CH), jnp.int32),
            pltpu.VMEM((NCH, CH), jnp.int32),
            pltpu.VMEM((NCH, CH), jnp.int32),
            pltpu.VMEM((NCH, CH), jnp.int32),
            pltpu.VMEM((ROWS, 32), jnp.float32),
            pltpu.VMEM((ROWS, 32), jnp.float32),
            pltpu.VMEM((ROWS, 32), jnp.float32),
            pltpu.VMEM((ROWS, 32), jnp.float32),
            pltpu.VMEM_SHARED((N, 32), jnp.float32),
            pltpu.SemaphoreType.DMA,
        ],
    )(h, se, de, so, do)


def _sc_gather_ts(ts16, se, de, so, do):
    return pl.kernel(
        _gather_ts_body,
        out_type=jax.ShapeDtypeStruct((P2, 64), jnp.float32),
        mesh=_SC_MESH,
        compiler_params=_SC_PARAMS,
        scratch_types=[
            pltpu.VMEM((NCH, CH), jnp.int32),
            pltpu.VMEM((NCH, CH), jnp.int32),
            pltpu.VMEM((NCH, CH), jnp.int32),
            pltpu.VMEM((NCH, CH), jnp.int32),
            pltpu.VMEM((ROWS, 16), jnp.float32),
            pltpu.VMEM((ROWS, 16), jnp.float32),
            pltpu.VMEM((ROWS, 16), jnp.float32),
            pltpu.VMEM((ROWS, 16), jnp.float32),
            pltpu.VMEM_SHARED((N, 16), jnp.float32),
            pltpu.SemaphoreType.DMA,
        ],
    )(ts16, se, de, so, do)


def _sc_scatter(ea2, de, do, zeros_n64):
    return pl.kernel(
        _scatter_body,
        out_type=jax.ShapeDtypeStruct((N, 128), jnp.float32),
        mesh=_SC_MESH,
        compiler_params=_SC_PARAMS,
        scratch_types=[
            pltpu.VMEM((NCH, CH), jnp.int32),
            pltpu.VMEM((NCH, CH), jnp.int32),
            pltpu.VMEM((ROWS, 64), jnp.float32),
            pltpu.VMEM((ROWS, 64), jnp.float32),
            pltpu.VMEM_SHARED((N, 64), jnp.float32),
            pltpu.SemaphoreType.DMA,
        ],
    )(ea2, de, do, zeros_n64)


BN = 1000
BP = 4000


def _enc_node_body(x_ref, w1, b1, w2, b2, wi, bi, h_ref, cn_ref):
    h1 = jnp.maximum(x_ref[...] @ w1[...] + b1[...], 0.0)
    h = h1 @ w2[...] + b2[...]
    h_ref[...] = h
    cn_ref[...] = h @ wi[...] + bi[...]


def _enc_edge_body(a2_ref, tg_ref, w1, b1, w2, b2, ea_ref, dt_ref):
    a2 = a2_ref[...]
    h1e = jnp.maximum(a2[:, 0:8] @ w1[...] + b1[...], 0.0)
    h1o = jnp.maximum(a2[:, 8:16] @ w1[...] + b1[...], 0.0)
    ea_ref[...] = jnp.concatenate(
        [h1e @ w2[...] + b2[...], h1o @ w2[...] + b2[...]], axis=1)
    tg = tg_ref[...]
    dt_ref[...] = jnp.concatenate(
        [tg[:, 16:17] - tg[:, 0:1], tg[:, 48:49] - tg[:, 32:33]], axis=1)


def _edge_mlp_body(g_ref, ea_ref, dt_ref, w1sde, wdt, b1, w2, b2,
                   out_ref):
    g = g_ref[...]
    ea = ea_ref[...]
    dt = dt_ref[...]
    w1 = w1sde[...]
    f_e = jnp.concatenate([g[:, 0:64], ea[:, 0:64]], axis=1)
    f_o = jnp.concatenate([g[:, 64:128], ea[:, 64:128]], axis=1)
    t_e = f_e @ w1 + dt[:, 0:1] * wdt[...] + b1[...]
    t_o = f_o @ w1 + dt[:, 1:2] * wdt[...] + b1[...]
    out_ref[...] = jnp.concatenate(
        [jnp.maximum(t_e, 0.0) @ w2[...] + b2[...],
         jnp.maximum(t_o, 0.0) @ w2[...] + b2[...]], axis=1)


def _node_mlp_body(h_ref, agga_ref, aggb_ref, cn_ref, wh, wa, w2, b2,
                   out_ref):
    a = agga_ref[...]
    b = aggb_ref[...]
    agg = (a[:, 0:64] + a[:, 64:128]) + (b[:, 0:64] + b[:, 64:128])
    pre = h_ref[...] @ wh[...] + agg @ wa[...] + cn_ref[...]
    out_ref[...] = jnp.maximum(pre, 0.0) @ w2[...] + b2[...]


def _cls_body(ea_ref, w1, b1, w2, b2, out_ref):
    ea = ea_ref[...]
    u_e = jnp.maximum(ea[:, 0:64] @ w1[...] + b1[...], 0.0)
    u_o = jnp.maximum(ea[:, 64:128] @ w1[...] + b1[...], 0.0)
    out_ref[...] = jax.nn.sigmoid(jnp.concatenate(
        [u_e @ w2[...] + b2[...], u_o @ w2[...] + b2[...]], axis=1))


def _full(shape):
    return pl.BlockSpec(shape, lambda i: tuple(0 for _ in shape))


def _tc_enc_node(x, ne_W1, ne_b1, ne_W2, ne_b2, nmWi, nm_b1):
    return pl.pallas_call(
        _enc_node_body,
        grid=(N // BN,),
        in_specs=[pl.BlockSpec((BN, 512), lambda i: (i, 0)),
                  _full((512, 256)), _full((1, 256)),
                  _full((256, 32)), _full((1, 32)),
                  _full((32, 32)), _full((1, 32))],
        out_specs=[pl.BlockSpec((BN, 32), lambda i: (i, 0)),
                   pl.BlockSpec((BN, 32), lambda i: (i, 0))],
        out_shape=[jax.ShapeDtypeStruct((N, 32), jnp.float32),
                   jax.ShapeDtypeStruct((N, 32), jnp.float32)],
    )(x, ne_W1, ne_b1, ne_W2, ne_b2, nmWi, nm_b1)


def _tc_enc_edge(attr2, tg, ee_W1p, ee_b1, ee_W2, ee_b2):
    return pl.pallas_call(
        _enc_edge_body,
        grid=(P2 // BP,),
        in_specs=[pl.BlockSpec((BP, 16), lambda i: (i, 0)),
                  pl.BlockSpec((BP, 64), lambda i: (i, 0)),
                  _full((8, 32)), _full((1, 32)),
                  _full((32, 64)), _full((1, 64))],
        out_specs=[pl.BlockSpec((BP, 128), lambda i: (i, 0)),
                   pl.BlockSpec((BP, 2), lambda i: (i, 0))],
        out_shape=[jax.ShapeDtypeStruct((P2, 128), jnp.float32),
                   jax.ShapeDtypeStruct((P2, 2), jnp.float32)],
    )(attr2, tg, ee_W1p, ee_b1, ee_W2, ee_b2)


def _tc_edge_mlp(g2, ea2, dt2, w1sde, wdt, b1, w2, b2):
    return pl.pallas_call(
        _edge_mlp_body,
        grid=(P2 // BP,),
        in_specs=[pl.BlockSpec((BP, 128), lambda i: (i, 0)),
                  pl.BlockSpec((BP, 128), lambda i: (i, 0)),
                  pl.BlockSpec((BP, 2), lambda i: (i, 0)),
                  _full((128, 64)), _full((1, 64)),
                  _full((1, 64)), _full((64, 64)), _full((1, 64))],
        out_specs=pl.BlockSpec((BP, 128), lambda i: (i, 0)),
        out_shape=jax.ShapeDtypeStruct((P2, 128), jnp.float32),
    )(g2, ea2, dt2, w1sde, wdt, b1, w2, b2)


def _tc_node_mlp(h, agga, aggb, cn, wh, wa, w2, b2):
    return pl.pallas_call(
        _node_mlp_body,
        grid=(N // BN,),
        in_specs=[pl.BlockSpec((BN, 32), lambda i: (i, 0)),
                  pl.BlockSpec((BN, 128), lambda i: (i, 0)),
                  pl.BlockSpec((BN, 128), lambda i: (i, 0)),
                  pl.BlockSpec((BN, 32), lambda i: (i, 0)),
                  _full((32, 32)), _full((64, 32)),
                  _full((32, 32)), _full((1, 32))],
        out_specs=pl.BlockSpec((BN, 32), lambda i: (i, 0)),
        out_shape=jax.ShapeDtypeStruct((N, 32), jnp.float32),
    )(h, agga, aggb, cn, wh, wa, w2, b2)


def _tc_cls(ea2, ec_W1, ec_b1, ec_W2, ec_b2):
    return pl.pallas_call(
        _cls_body,
        grid=(P2 // BP,),
        in_specs=[pl.BlockSpec((BP, 128), lambda i: (i, 0)),
                  _full((64, 64)), _full((1, 64)),
                  _full((64, 1)), _full((1, 1))],
        out_specs=pl.BlockSpec((BP, 2), lambda i: (i, 0)),
        out_shape=jax.ShapeDtypeStruct((P2, 2), jnp.float32),
    )(ea2, ec_W1, ec_b1, ec_W2, ec_b2)


def kernel(x, edge_index, edge_attr, node_timestamps, batch, initial_x,
           ee_W1, ee_b1, ee_W2, ee_b2,
           ne_W1, ne_b1, ne_W2, ne_b2,
           em_W1, em_b1, em_W2, em_b2,
           nm_W1, nm_b1, nm_W2, nm_b2,
           ec_W1, ec_b1, ec_W2, ec_b2):
    src = edge_index[0]
    dst = edge_index[1]
    rsh = lambda a: a.reshape(NW, NCH, CH)
    idx_a = (rsh(src[:P2]), rsh(dst[:P2]),
             rsh(src[P:P + P2]), rsh(dst[P:P + P2]))
    idx_b = (rsh(src[P2:P]), rsh(dst[P2:P]),
             rsh(src[P + P2:]), rsh(dst[P + P2:]))

    attr_p = jnp.pad(edge_attr, ((0, 0), (0, 2)))
    attr2 = jnp.concatenate([attr_p[:P], attr_p[P:]], axis=1)
    attr2a, attr2b = attr2[:P2], attr2[P2:]
    ee_W1p = jnp.pad(ee_W1, ((0, 2), (0, 0)))
    ts16 = jnp.pad(node_timestamps[:, None], ((0, 0), (0, 15)))
    w1sde = em_W1[0:128]
    wdt = em_W1[128:129]
    nmWh = nm_W1[0:32]
    nmWa = nm_W1[32:96]
    nmWi = nm_W1[96:128]
    zeros_n64 = jnp.zeros((N, 64), jnp.float32)

    r = lambda b: b[None, :]

    h, cn = _tc_enc_node(x, ne_W1, r(ne_b1), ne_W2, r(ne_b2), nmWi, r(nm_b1))
    tga = _sc_gather_ts(ts16, *idx_a)
    tgb = _sc_gather_ts(ts16, *idx_b)
    ea2a, dt2a = _tc_enc_edge(attr2a, tga, ee_W1p, r(ee_b1), ee_W2, r(ee_b2))
    ea2b, dt2b = _tc_enc_edge(attr2b, tgb, ee_W1p, r(ee_b1), ee_W2, r(ee_b2))

    for _ in range(12):
        g2a = _sc_gather_h(h, *idx_a)
        g2b = _sc_gather_h(h, *idx_b)
        ea2a = _tc_edge_mlp(g2a, ea2a, dt2a, w1sde, wdt, r(em_b1), em_W2,
                            r(em_b2))
        agga = _sc_scatter(ea2a, idx_a[1], idx_a[3], zeros_n64)
        ea2b = _tc_edge_mlp(g2b, ea2b, dt2b, w1sde, wdt, r(em_b1), em_W2,
                            r(em_b2))
        aggb = _sc_scatter(ea2b, idx_b[1], idx_b[3], zeros_n64)
        h = _tc_node_mlp(h, agga, aggb, cn, nmWh, nmWa, nm_W2, r(nm_b2))

    outa = _tc_cls(ea2a, ec_W1, r(ec_b1), ec_W2, r(ec_b2))
    outb = _tc_cls(ea2b, ec_W1, r(ec_b1), ec_W2, r(ec_b2))
    return jnp.concatenate([outa[:, 0:1], outb[:, 0:1],
                            outa[:, 1:2], outb[:, 1:2]], axis=0)
